# Initial kernel scaffold; baseline (speedup 1.0000x reference)
import jax
import jax.numpy as jnp
from jax.experimental import pallas as pl


def kernel(x, edge_index, edge_attr, WQ, bQ, WK, WV, WE, bE, Aw, VeRow):
    raise NotImplementedError("write your pallas kernel here")


# speedup vs baseline: 8.2705x; 8.2705x over previous
import functools

import jax
import jax.numpy as jnp
from jax import lax
from jax.experimental import pallas as pl
from jax.experimental.pallas import tpu as pltpu
from jax.experimental.pallas import tpu_sc as plsc

N = 10000
IN = 256
H = 8
D = 32
EG = 160000

NP = 10240
EGP = 163840
CH = 128
NC = 2
NS = 16
L = 16
CPT32 = EGP // (CH * NC * NS)
CPT16 = EGP // (CH * NS)
RPN = NP // NS
EB = 512
NB = 512
AW = 128
ACCN = 10112
RPA = ACCN // NS
NB6 = 632


def _vmesh():
    return plsc.VectorSubcoreMesh(core_axis_name="c", subcore_axis_name="s",
                                  num_cores=NC, num_subcores=NS)


def _qkv_body(x_ref, wq_ref, wk_ref, wv_ref, bq_ref, q_ref, k_ref, v_ref):
    xb = x_ref[...]
    q_ref[...] = jnp.dot(xb, wq_ref[...], preferred_element_type=jnp.float32) + bq_ref[...]
    k_ref[...] = jnp.dot(xb, wk_ref[...], preferred_element_type=jnp.float32)
    v_ref[...] = jnp.dot(xb, wv_ref[...], preferred_element_type=jnp.float32)


def _qkv(xp, WQ, WK, WV, bQ):
    return pl.pallas_call(
        _qkv_body,
        grid=(NP // NB,),
        in_specs=[
            pl.BlockSpec((NB, IN), lambda i: (i, 0)),
            pl.BlockSpec((IN, H * D), lambda i: (0, 0)),
            pl.BlockSpec((IN, H * D), lambda i: (0, 0)),
            pl.BlockSpec((IN, H * D), lambda i: (0, 0)),
            pl.BlockSpec((1, H * D), lambda i: (0, 0)),
        ],
        out_specs=[
            pl.BlockSpec((NB, H * D), lambda i: (i, 0)),
            pl.BlockSpec((NB, H * D), lambda i: (i, 0)),
            pl.BlockSpec((NB, H * D), lambda i: (i, 0)),
        ],
        out_shape=[
            jax.ShapeDtypeStruct((NP, H * D), jnp.float32),
            jax.ShapeDtypeStruct((NP, H * D), jnp.float32),
            jax.ShapeDtypeStruct((NP, H * D), jnp.float32),
        ],
    )(xp, WQ, WK, WV, bQ.reshape(1, H * D))


@functools.cache
def _build_gather_kqv():
    @functools.partial(
        pl.kernel,
        out_type=[
            jax.ShapeDtypeStruct((EGP, H * D), jnp.float32),
            jax.ShapeDtypeStruct((EGP, H * D), jnp.float32),
            jax.ShapeDtypeStruct((EGP, H * D), jnp.float32),
        ],
        mesh=_vmesh(),
        scratch_types=[
            pltpu.VMEM((CH,), jnp.int32),
            pltpu.VMEM((CH,), jnp.int32),
            pltpu.VMEM((CH, H * D), jnp.float32),
            pltpu.VMEM((CH, H * D), jnp.float32),
            pltpu.VMEM((CH, H * D), jnp.float32),
            pltpu.SemaphoreType.DMA,
            pltpu.SemaphoreType.DMA,
            pltpu.SemaphoreType.DMA,
        ],
        compiler_params=pltpu.CompilerParams(needs_layout_passes=False),
    )
    def _gather_kqv(km, qm, vm, srcp, dstp, ks_out, qd_out, vs_out,
                    idxs, idxd, bufk, bufq, bufv, semk, semq, semv):
        cid = lax.axis_index("c")
        sid = lax.axis_index("s")
        wid = sid * NC + cid

        def body(j, carry):
            base = (wid * CPT32 + j) * CH
            pltpu.sync_copy(srcp.at[pl.ds(base, CH)], idxs)
            pltpu.sync_copy(dstp.at[pl.ds(base, CH)], idxd)
            ck = pltpu.async_copy(km.at[idxs], bufk, semk)
            cq = pltpu.async_copy(qm.at[idxd], bufq, semq)
            cv = pltpu.async_copy(vm.at[idxs], bufv, semv)
            ck.wait()
            cq.wait()
            cv.wait()
            pltpu.sync_copy(bufk, ks_out.at[pl.ds(base, CH)])
            pltpu.sync_copy(bufq, qd_out.at[pl.ds(base, CH)])
            pltpu.sync_copy(bufv, vs_out.at[pl.ds(base, CH)])
            return carry

        lax.fori_loop(0, CPT32, body, 0)

    return _gather_kqv


def _edge_body(ea_ref, ks_ref, qd_ref, vs_ref, we_ref, be_ref, awm_ref,
               weo_ref, ett_ref, vstt_ref, exh_ref):
    ee = jnp.dot(ea_ref[...], we_ref[...], preferred_element_type=jnp.float32)
    ee = ee + be_ref[...]
    s0 = ks_ref[...] + qd_ref[...]
    vs = vs_ref[...]
    cols = []
    for h in range(H):
        t = s0[:, h * D:(h + 1) * D] * ee[:, 2 * h * D:2 * h * D + D]
        sc = jnp.sqrt(jax.nn.relu(t)) - jnp.sqrt(jax.nn.relu(-t))
        sc = sc + ee[:, 2 * h * D + D:2 * (h + 1) * D]
        weo_ref[:, h * D:(h + 1) * D] = sc
        ett_ref[0, h * D:(h + 1) * D, :] = sc.T
        vstt_ref[0, h * D:(h + 1) * D, :] = vs[:, h * D:(h + 1) * D].T
        cols.append(sc)
    score = jnp.concatenate(cols, axis=1)
    s = jnp.dot(score, awm_ref[...], preferred_element_type=jnp.float32)
    s = jnp.clip(s, -5.0, 5.0)
    ex = jnp.exp(s)
    exh_ref[...] = ex.T[:H]


def _edge_scores(eap, KS, QD, VS, WE, bE, AwM):
    return pl.pallas_call(
        _edge_body,
        grid=(EGP // EB,),
        in_specs=[
            pl.BlockSpec((EB, IN), lambda i: (i, 0)),
            pl.BlockSpec((EB, H * D), lambda i: (i, 0)),
            pl.BlockSpec((EB, H * D), lambda i: (i, 0)),
            pl.BlockSpec((EB, H * D), lambda i: (i, 0)),
            pl.BlockSpec((IN, 2 * H * D), lambda i: (0, 0)),
            pl.BlockSpec((1, 2 * H * D), lambda i: (0, 0)),
            pl.BlockSpec((H * D, L), lambda i: (0, 0)),
        ],
        out_specs=[
            pl.BlockSpec((EB, H * D), lambda i: (i, 0)),
            pl.BlockSpec((1, H * D, EB), lambda i: (i, 0, 0)),
            pl.BlockSpec((1, H * D, EB), lambda i: (i, 0, 0)),
            pl.BlockSpec((H, EB), lambda i: (0, i)),
        ],
        out_shape=[
            jax.ShapeDtypeStruct((EGP, H * D), jnp.float32),
            jax.ShapeDtypeStruct((EGP // EB, H * D, EB), jnp.float32),
            jax.ShapeDtypeStruct((EGP // EB, H * D, EB), jnp.float32),
            jax.ShapeDtypeStruct((H, EGP), jnp.float32),
        ],
    )(eap, KS, QD, VS, WE, bE.reshape(1, 2 * H * D), AwM)


@functools.cache
def _build_aggregate(r):
    CPB = EB // CH
    BPT = CPT16 // CPB

    @functools.partial(
        pl.kernel,
        out_type=jax.ShapeDtypeStruct((NC * ACCN, AW), jnp.float32),
        mesh=_vmesh(),
        scratch_types=[
            pltpu.VMEM((CH,), jnp.int32),
            pltpu.VMEM((D * EB,), jnp.float32),
            pltpu.VMEM((D * EB,), jnp.float32),
            pltpu.VMEM((CH,), jnp.float32),
            pltpu.VMEM((CH, AW), jnp.float32),
            pltpu.VMEM_SHARED((ACCN, AW), jnp.float32),
        ],
        compiler_params=pltpu.CompilerParams(needs_layout_passes=False),
    )
    def _aggregate(ettf, vsttf, exh2, dstp, znpw, acc_out,
                   idxd, vbuf, etbuf, exbuf, msg, acc):
        cid = lax.axis_index("c")
        sid = lax.axis_index("s")
        h = cid * (H // NC) + r
        hoff_e = h * EGP
        hrow = h * D

        pltpu.sync_copy(znpw.at[pl.ds(0, CH)], msg)
        pltpu.sync_copy(znpw.at[pl.ds(sid * RPA, RPA)],
                        acc.at[pl.ds(sid * RPA, RPA)])
        plsc.subcore_barrier()

        def batch(jj, carry0):
            bb = sid * BPT + jj
            foff = (bb * (H * D) + hrow) * EB
            pltpu.sync_copy(vsttf.at[pl.ds(foff, D * EB)], vbuf)
            pltpu.sync_copy(ettf.at[pl.ds(foff, D * EB)], etbuf)

            def chunk(cc, carry):
                base = bb * EB + cc * CH
                pltpu.sync_copy(dstp.at[pl.ds(base, CH)], idxd)
                pltpu.sync_copy(exh2.at[pl.ds(hoff_e + base, CH)], exbuf)
                for g in range(CH // L):
                    eids = lax.iota(jnp.int32, L) + g * L
                    ex16 = exbuf[pl.ds(g * L, L)]
                    eb16 = eids + cc * CH
                    for jc in range(2 * D):
                        srcf = vbuf if jc < D else etbuf
                        cvec = plsc.load_gather(srcf, [eb16 + (jc % D) * EB])
                        plsc.store_scatter(
                            msg, [eids, jnp.full((L,), jc, jnp.int32)],
                            cvec * ex16)
                    plsc.store_scatter(
                        msg, [eids, jnp.full((L,), 2 * D, jnp.int32)], ex16)
                pltpu.sync_copy(msg, acc.at[idxd], add=True)
                return carry

            lax.fori_loop(0, CPB, chunk, 0)
            return carry0

        lax.fori_loop(0, BPT, batch, 0)
        plsc.subcore_barrier()
        pltpu.sync_copy(acc.at[pl.ds(sid * RPA, RPA)],
                        acc_out.at[pl.ds(cid * ACCN + sid * RPA, RPA)])

    return _aggregate


def _final_body(a0_ref, a1_ref, a2_ref, a3_ref, ver_ref, o_ref):
    refs = [a0_ref, a1_ref, a2_ref, a3_ref]
    for c in range(NC):
        for r in range(H // NC):
            h = c * (H // NC) + r
            a = refs[r][c]
            av = a[:, :D]
            ae = a[:, D:2 * D]
            ssn = a[:, 2 * D:2 * D + 1]
            o_ref[:, h * D:(h + 1) * D] = (av + jnp.dot(
                ae, ver_ref[h], preferred_element_type=jnp.float32)
            ) / (ssn + 1e-16)


def _final(parts, VeR):
    spec = pl.BlockSpec((NC, NB6, AW), lambda i: (0, i, 0))
    return pl.pallas_call(
        _final_body,
        grid=(ACCN // NB6,),
        in_specs=[spec, spec, spec, spec,
                  pl.BlockSpec((H, D, D), lambda i: (0, 0, 0))],
        out_specs=pl.BlockSpec((NB6, H * D), lambda i: (i, 0)),
        out_shape=jax.ShapeDtypeStruct((ACCN, H * D), jnp.float32),
    )(*[p.reshape(NC, ACCN, AW) for p in parts], VeR)


def kernel(x, edge_index, edge_attr, WQ, bQ, WK, WV, WE, bE, Aw, VeRow):
    f32 = jnp.float32
    xp = jnp.zeros((NP, IN), f32).at[:N].set(x.astype(f32))
    src = jnp.zeros((EGP,), jnp.int32).at[:EG].set(edge_index[0].astype(jnp.int32))
    dst = jnp.full((EGP,), N, jnp.int32).at[:EG].set(edge_index[1].astype(jnp.int32))
    eap = jnp.zeros((EGP, IN), f32).at[:EG].set(edge_attr.astype(f32))

    A = Aw[:, :, 0].astype(f32)
    E8 = jnp.eye(H, L, dtype=f32)
    AwM = (A.T[:, :, None] * E8[:, None, :]).reshape(H * D, L)
    VeR = VeRow.transpose(1, 0, 2).astype(f32)

    Qm, Km, Vm = _qkv(xp, WQ.astype(f32), WK.astype(f32), WV.astype(f32),
                      bQ.astype(f32))
    KS, QD, VS = _build_gather_kqv()(Km, Qm, Vm, src, dst)
    WEo, ETt, VSTt, EXh = _edge_scores(eap, KS, QD, VS, WE.astype(f32),
                                       bE.astype(f32), AwM)
    exh2 = EXh.reshape(H * EGP)
    ettf = ETt.reshape(EGP * H * D)
    vsttf = VSTt.reshape(EGP * H * D)
    znpw = jnp.zeros((ACCN, AW), f32)
    parts = [_build_aggregate(r)(ettf, vsttf, exh2, dst, znpw)
             for r in range(H // NC)]
    O = _final(parts, VeR)

    wV = O[:N].reshape(N, H, D)
    wE = WEo[:EG]
    return (wV, wE)


# speedup vs baseline: 8.5908x; 1.0387x over previous
import functools

import jax
import jax.numpy as jnp
from jax import lax
from jax.experimental import pallas as pl
from jax.experimental.pallas import tpu as pltpu
from jax.experimental.pallas import tpu_sc as plsc

N = 10000
IN = 256
H = 8
D = 32
EG = 160000

NP = 10240
EGP = 163840
CH = 128
NC = 2
NS = 16
L = 16
CPT32 = EGP // (CH * NC * NS)
CPT16 = EGP // (CH * NS)
RPN = NP // NS
EB = 512
NB = 512
AW = 128
ACCN = 10112
RPA = ACCN // NS
NB6 = 632


def _vmesh():
    return plsc.VectorSubcoreMesh(core_axis_name="c", subcore_axis_name="s",
                                  num_cores=NC, num_subcores=NS)


def _qkv_body(x_ref, wq_ref, wk_ref, wv_ref, bq_ref, q_ref, k_ref, v_ref):
    xb = x_ref[...]
    q_ref[...] = jnp.dot(xb, wq_ref[...], preferred_element_type=jnp.float32) + bq_ref[...]
    k_ref[...] = jnp.dot(xb, wk_ref[...], preferred_element_type=jnp.float32)
    v_ref[...] = jnp.dot(xb, wv_ref[...], preferred_element_type=jnp.float32)


def _qkv(xp, WQ, WK, WV, bQ):
    return pl.pallas_call(
        _qkv_body,
        grid=(NP // NB,),
        in_specs=[
            pl.BlockSpec((NB, IN), lambda i: (i, 0)),
            pl.BlockSpec((IN, H * D), lambda i: (0, 0)),
            pl.BlockSpec((IN, H * D), lambda i: (0, 0)),
            pl.BlockSpec((IN, H * D), lambda i: (0, 0)),
            pl.BlockSpec((1, H * D), lambda i: (0, 0)),
        ],
        out_specs=[
            pl.BlockSpec((NB, H * D), lambda i: (i, 0)),
            pl.BlockSpec((NB, H * D), lambda i: (i, 0)),
            pl.BlockSpec((NB, H * D), lambda i: (i, 0)),
        ],
        out_shape=[
            jax.ShapeDtypeStruct((NP, H * D), jnp.float32),
            jax.ShapeDtypeStruct((NP, H * D), jnp.float32),
            jax.ShapeDtypeStruct((NP, H * D), jnp.float32),
        ],
    )(xp, WQ, WK, WV, bQ.reshape(1, H * D))


@functools.cache
def _build_gather_kqv():
    @functools.partial(
        pl.kernel,
        out_type=[
            jax.ShapeDtypeStruct((EGP, H * D), jnp.float32),
            jax.ShapeDtypeStruct((EGP, H * D), jnp.float32),
            jax.ShapeDtypeStruct((EGP, H * D), jnp.float32),
        ],
        mesh=_vmesh(),
        scratch_types=[
            pltpu.VMEM((CH,), jnp.int32),
            pltpu.VMEM((CH,), jnp.int32),
            pltpu.VMEM((CH, H * D), jnp.float32),
            pltpu.VMEM((CH, H * D), jnp.float32),
            pltpu.VMEM((CH, H * D), jnp.float32),
            pltpu.SemaphoreType.DMA,
            pltpu.SemaphoreType.DMA,
            pltpu.SemaphoreType.DMA,
        ],
        compiler_params=pltpu.CompilerParams(needs_layout_passes=False),
    )
    def _gather_kqv(km, qm, vm, srcp, dstp, ks_out, qd_out, vs_out,
                    idxs, idxd, bufk, bufq, bufv, semk, semq, semv):
        cid = lax.axis_index("c")
        sid = lax.axis_index("s")
        wid = sid * NC + cid

        def body(j, carry):
            base = (wid * CPT32 + j) * CH
            pltpu.sync_copy(srcp.at[pl.ds(base, CH)], idxs)
            pltpu.sync_copy(dstp.at[pl.ds(base, CH)], idxd)
            ck = pltpu.async_copy(km.at[idxs], bufk, semk)
            cq = pltpu.async_copy(qm.at[idxd], bufq, semq)
            cv = pltpu.async_copy(vm.at[idxs], bufv, semv)
            ck.wait()
            cq.wait()
            cv.wait()
            pltpu.sync_copy(bufk, ks_out.at[pl.ds(base, CH)])
            pltpu.sync_copy(bufq, qd_out.at[pl.ds(base, CH)])
            pltpu.sync_copy(bufv, vs_out.at[pl.ds(base, CH)])
            return carry

        lax.fori_loop(0, CPT32, body, 0)

    return _gather_kqv


def _edge_body(ea_ref, ks_ref, qd_ref, vs_ref, we_ref, be_ref, awm_ref,
               weo_ref, ett_ref, vstt_ref, exh_ref):
    ee = jnp.dot(ea_ref[...], we_ref[...], preferred_element_type=jnp.float32)
    ee = ee + be_ref[...]
    s0 = ks_ref[...] + qd_ref[...]
    vs = vs_ref[...]
    cols = []
    for h in range(H):
        t = s0[:, h * D:(h + 1) * D] * ee[:, 2 * h * D:2 * h * D + D]
        sc = jnp.sqrt(jax.nn.relu(t)) - jnp.sqrt(jax.nn.relu(-t))
        sc = sc + ee[:, 2 * h * D + D:2 * (h + 1) * D]
        weo_ref[:, h * D:(h + 1) * D] = sc
        ett_ref[0, h * D:(h + 1) * D, :] = sc.T
        vstt_ref[0, h * D:(h + 1) * D, :] = vs[:, h * D:(h + 1) * D].T
        cols.append(sc)
    score = jnp.concatenate(cols, axis=1)
    s = jnp.dot(score, awm_ref[...], preferred_element_type=jnp.float32)
    s = jnp.clip(s, -5.0, 5.0)
    ex = jnp.exp(s)
    exh_ref[...] = ex.T[:H]


def _edge_scores(eap, KS, QD, VS, WE, bE, AwM):
    return pl.pallas_call(
        _edge_body,
        grid=(EGP // EB,),
        in_specs=[
            pl.BlockSpec((EB, IN), lambda i: (i, 0)),
            pl.BlockSpec((EB, H * D), lambda i: (i, 0)),
            pl.BlockSpec((EB, H * D), lambda i: (i, 0)),
            pl.BlockSpec((EB, H * D), lambda i: (i, 0)),
            pl.BlockSpec((IN, 2 * H * D), lambda i: (0, 0)),
            pl.BlockSpec((1, 2 * H * D), lambda i: (0, 0)),
            pl.BlockSpec((H * D, L), lambda i: (0, 0)),
        ],
        out_specs=[
            pl.BlockSpec((EB, H * D), lambda i: (i, 0)),
            pl.BlockSpec((1, H * D, EB), lambda i: (i, 0, 0)),
            pl.BlockSpec((1, H * D, EB), lambda i: (i, 0, 0)),
            pl.BlockSpec((H, EB), lambda i: (0, i)),
        ],
        out_shape=[
            jax.ShapeDtypeStruct((EGP, H * D), jnp.float32),
            jax.ShapeDtypeStruct((EGP // EB, H * D, EB), jnp.float32),
            jax.ShapeDtypeStruct((EGP // EB, H * D, EB), jnp.float32),
            jax.ShapeDtypeStruct((H, EGP), jnp.float32),
        ],
    )(eap, KS, QD, VS, WE, bE.reshape(1, 2 * H * D), AwM)


@functools.cache
def _build_aggregate(r):
    CPB = EB // CH
    BPT = CPT16 // CPB

    @functools.partial(
        pl.kernel,
        out_type=jax.ShapeDtypeStruct((NC * ACCN, AW), jnp.float32),
        mesh=_vmesh(),
        scratch_types=[
            pltpu.VMEM((CH,), jnp.int32),
            pltpu.VMEM((D * EB,), jnp.float32),
            pltpu.VMEM((D * EB,), jnp.float32),
            pltpu.VMEM((CH,), jnp.float32),
            pltpu.VMEM((CH, AW), jnp.float32),
            pltpu.VMEM_SHARED((ACCN, AW), jnp.float32),
            pltpu.SemaphoreType.DMA,
            pltpu.SemaphoreType.DMA,
        ],
        compiler_params=pltpu.CompilerParams(needs_layout_passes=False),
    )
    def _aggregate(ettf, vsttf, exh2, dstp, znpw, acc_out,
                   idxd, vbuf, etbuf, exbuf, msg, acc, semA, semB):
        cid = lax.axis_index("c")
        sid = lax.axis_index("s")
        h = cid * (H // NC) + r
        hoff_e = h * EGP
        hrow = h * D

        pltpu.sync_copy(znpw.at[pl.ds(0, CH)], msg)
        pltpu.sync_copy(znpw.at[pl.ds(sid * RPA, RPA)],
                        acc.at[pl.ds(sid * RPA, RPA)])
        plsc.subcore_barrier()

        def batch(jj, carry0):
            bb = sid * BPT + jj
            foff = (bb * (H * D) + hrow) * EB
            cv = pltpu.async_copy(vsttf.at[pl.ds(foff, D * EB)], vbuf, semA)
            ce = pltpu.async_copy(ettf.at[pl.ds(foff, D * EB)], etbuf, semB)
            cv.wait()
            ce.wait()

            def chunk(cc, carry):
                base = bb * EB + cc * CH
                ci = pltpu.async_copy(dstp.at[pl.ds(base, CH)], idxd, semA)
                cx = pltpu.async_copy(exh2.at[pl.ds(hoff_e + base, CH)], exbuf, semB)
                ci.wait()
                cx.wait()
                for g in range(CH // L):
                    eids = lax.iota(jnp.int32, L) + g * L
                    ex16 = exbuf[pl.ds(g * L, L)]
                    eb16 = eids + cc * CH
                    for jc in range(2 * D):
                        srcf = vbuf if jc < D else etbuf
                        cvec = plsc.load_gather(srcf, [eb16 + (jc % D) * EB])
                        plsc.store_scatter(
                            msg, [eids, jnp.full((L,), jc, jnp.int32)],
                            cvec * ex16)
                    plsc.store_scatter(
                        msg, [eids, jnp.full((L,), 2 * D, jnp.int32)], ex16)
                pltpu.sync_copy(msg, acc.at[idxd], add=True)
                return carry

            lax.fori_loop(0, CPB, chunk, 0)
            return carry0

        lax.fori_loop(0, BPT, batch, 0)
        plsc.subcore_barrier()
        pltpu.sync_copy(acc.at[pl.ds(sid * RPA, RPA)],
                        acc_out.at[pl.ds(cid * ACCN + sid * RPA, RPA)])

    return _aggregate


def _final_body(a0_ref, a1_ref, a2_ref, a3_ref, ver_ref, o_ref):
    refs = [a0_ref, a1_ref, a2_ref, a3_ref]
    for c in range(NC):
        for r in range(H // NC):
            h = c * (H // NC) + r
            a = refs[r][c]
            av = a[:, :D]
            ae = a[:, D:2 * D]
            ssn = a[:, 2 * D:2 * D + 1]
            o_ref[:, h * D:(h + 1) * D] = (av + jnp.dot(
                ae, ver_ref[h], preferred_element_type=jnp.float32)
            ) / (ssn + 1e-16)


def _final(parts, VeR):
    spec = pl.BlockSpec((NC, NB6, AW), lambda i: (0, i, 0))
    return pl.pallas_call(
        _final_body,
        grid=(ACCN // NB6,),
        in_specs=[spec, spec, spec, spec,
                  pl.BlockSpec((H, D, D), lambda i: (0, 0, 0))],
        out_specs=pl.BlockSpec((NB6, H * D), lambda i: (i, 0)),
        out_shape=jax.ShapeDtypeStruct((ACCN, H * D), jnp.float32),
    )(*[p.reshape(NC, ACCN, AW) for p in parts], VeR)


def kernel(x, edge_index, edge_attr, WQ, bQ, WK, WV, WE, bE, Aw, VeRow):
    f32 = jnp.float32
    xp = jnp.zeros((NP, IN), f32).at[:N].set(x.astype(f32))
    src = jnp.zeros((EGP,), jnp.int32).at[:EG].set(edge_index[0].astype(jnp.int32))
    dst = jnp.full((EGP,), N, jnp.int32).at[:EG].set(edge_index[1].astype(jnp.int32))
    eap = jnp.zeros((EGP, IN), f32).at[:EG].set(edge_attr.astype(f32))

    A = Aw[:, :, 0].astype(f32)
    E8 = jnp.eye(H, L, dtype=f32)
    AwM = (A.T[:, :, None] * E8[:, None, :]).reshape(H * D, L)
    VeR = VeRow.transpose(1, 0, 2).astype(f32)

    Qm, Km, Vm = _qkv(xp, WQ.astype(f32), WK.astype(f32), WV.astype(f32),
                      bQ.astype(f32))
    KS, QD, VS = _build_gather_kqv()(Km, Qm, Vm, src, dst)
    WEo, ETt, VSTt, EXh = _edge_scores(eap, KS, QD, VS, WE.astype(f32),
                                       bE.astype(f32), AwM)
    exh2 = EXh.reshape(H * EGP)
    ettf = ETt.reshape(EGP * H * D)
    vsttf = VSTt.reshape(EGP * H * D)
    znpw = jnp.zeros((ACCN, AW), f32)
    parts = [_build_aggregate(r)(ettf, vsttf, exh2, dst, znpw)
             for r in range(H // NC)]
    O = _final(parts, VeR)

    wV = O[:N].reshape(N, H, D)
    wE = WEo[:EG]
    return (wV, wE)
